# baseline (speedup 1.0000x reference)
import functools

import jax
import jax.numpy as jnp
from jax import lax
from jax.experimental import pallas as pl
from jax.experimental.pallas import tpu as pltpu
from jax.experimental.pallas import tpu_sc as plsc

_INDEX = 10

_mesh = plsc.VectorSubcoreMesh(core_axis_name="c", subcore_axis_name="s")


@functools.partial(
    pl.kernel,
    mesh=_mesh,
    out_type=[
        jax.ShapeDtypeStruct((8, 32, 8, 128), jnp.float32),
        jax.ShapeDtypeStruct((8, 32, 8, 128), jnp.float32),
    ],
    scratch_types=[pltpu.VMEM((16, 8, 128), jnp.float32)],
)
def _sc_slice(x0v, x1v, o0, o1, buf):
    wid = lax.axis_index("s") * 2 + lax.axis_index("c")
    q = wid // 2
    t8 = q // 2
    t32 = (q % 2) * 16

    @pl.when(wid % 2 == 0)
    def _():
        pltpu.sync_copy(x0v.at[_INDEX, t8, pl.ds(t32, 16)], buf)
        pltpu.sync_copy(buf, o0.at[t8, pl.ds(t32, 16)])

    @pl.when(wid % 2 == 1)
    def _():
        pltpu.sync_copy(x1v.at[_INDEX, t8, pl.ds(t32, 16)], buf)
        pltpu.sync_copy(buf, o1.at[t8, pl.ds(t32, 16)])


def kernel(x0, x1):
    x0v = x0.reshape(32, 128, 200, 8, 8).transpose(2, 3, 0, 4, 1)
    x1v = x1.reshape(32, 128, 200, 8, 8).transpose(2, 3, 0, 4, 1)
    o0, o1 = _sc_slice(x0v, x1v)
    o0 = o0.transpose(1, 3, 0, 2).reshape(4096, 64)
    o1 = o1.transpose(1, 3, 0, 2).reshape(4096, 64)
    return o0, o1


# speedup vs baseline: 6.8121x; 6.8121x over previous
import jax
import jax.numpy as jnp
from jax.experimental import pallas as pl
from jax.experimental.pallas import tpu as pltpu

_INDEX = 10
_NCHUNK = 2
_CW = 4096 // _NCHUNK


def _slice_body(x0_hbm, x1_hbm, o0_hbm, o1_hbm, b0, b1, sin, sout):
    ins = []
    for t, (xh, bh) in enumerate(((x0_hbm, b0), (x1_hbm, b1))):
        for k in range(_NCHUNK):
            cp = pltpu.make_async_copy(
                xh.at[_INDEX, :, pl.ds(k * _CW, _CW)],
                bh.at[:, pl.ds(k * _CW, _CW)],
                sin.at[t * _NCHUNK + k],
            )
            cp.start()
            ins.append(cp)
    outs = []
    for t, (bh, oh) in enumerate(((b0, o0_hbm), (b1, o1_hbm))):
        for k in range(_NCHUNK):
            ins[t * _NCHUNK + k].wait()
            cp = pltpu.make_async_copy(
                bh.at[:, pl.ds(k * _CW, _CW)],
                oh.at[:, pl.ds(k * _CW, _CW)],
                sout.at[t * _NCHUNK + k],
            )
            cp.start()
            outs.append(cp)
    for cp in outs:
        cp.wait()


def kernel(x0, x1):
    B, S, D = x0.shape
    x0t = jnp.transpose(x0, (1, 2, 0))
    x1t = jnp.transpose(x1, (1, 2, 0))
    hbm = pl.BlockSpec(memory_space=pltpu.MemorySpace.HBM)
    o0t, o1t = pl.pallas_call(
        _slice_body,
        in_specs=[hbm, hbm],
        out_specs=[hbm, hbm],
        out_shape=[
            jax.ShapeDtypeStruct((D, B), x0.dtype),
            jax.ShapeDtypeStruct((D, B), x1.dtype),
        ],
        scratch_shapes=[
            pltpu.VMEM((D, B), x0.dtype),
            pltpu.VMEM((D, B), x1.dtype),
            pltpu.SemaphoreType.DMA((2 * _NCHUNK,)),
            pltpu.SemaphoreType.DMA((2 * _NCHUNK,)),
        ],
    )(x0t, x1t)
    return jnp.transpose(o0t, (1, 0)), jnp.transpose(o1t, (1, 0))


# speedup vs baseline: 6.8355x; 1.0034x over previous
import jax
import jax.numpy as jnp
from jax.experimental import pallas as pl
from jax.experimental.pallas import tpu as pltpu

_INDEX = 10
_NCHUNK = 4
_CW = 4096 // _NCHUNK


def _slice_body(x0_hbm, x1_hbm, o0_hbm, o1_hbm, b0, b1, sin, sout):
    ins = []
    for t, (xh, bh) in enumerate(((x0_hbm, b0), (x1_hbm, b1))):
        for k in range(_NCHUNK):
            cp = pltpu.make_async_copy(
                xh.at[_INDEX, :, pl.ds(k * _CW, _CW)],
                bh.at[:, pl.ds(k * _CW, _CW)],
                sin.at[t * _NCHUNK + k],
            )
            cp.start()
            ins.append(cp)
    outs = []
    for t, (bh, oh) in enumerate(((b0, o0_hbm), (b1, o1_hbm))):
        for k in range(_NCHUNK):
            ins[t * _NCHUNK + k].wait()
            cp = pltpu.make_async_copy(
                bh.at[:, pl.ds(k * _CW, _CW)],
                oh.at[:, pl.ds(k * _CW, _CW)],
                sout.at[t * _NCHUNK + k],
            )
            cp.start()
            outs.append(cp)
    for cp in outs:
        cp.wait()


def kernel(x0, x1):
    B, S, D = x0.shape
    x0t = jnp.transpose(x0, (1, 2, 0))
    x1t = jnp.transpose(x1, (1, 2, 0))
    hbm = pl.BlockSpec(memory_space=pltpu.MemorySpace.HBM)
    o0t, o1t = pl.pallas_call(
        _slice_body,
        in_specs=[hbm, hbm],
        out_specs=[hbm, hbm],
        out_shape=[
            jax.ShapeDtypeStruct((D, B), x0.dtype),
            jax.ShapeDtypeStruct((D, B), x1.dtype),
        ],
        scratch_shapes=[
            pltpu.VMEM((D, B), x0.dtype),
            pltpu.VMEM((D, B), x1.dtype),
            pltpu.SemaphoreType.DMA((2 * _NCHUNK,)),
            pltpu.SemaphoreType.DMA((2 * _NCHUNK,)),
        ],
    )(x0t, x1t)
    return jnp.transpose(o0t, (1, 0)), jnp.transpose(o1t, (1, 0))
